# baseline (speedup 1.0000x reference)
import functools

import jax
import jax.numpy as jnp
from jax import lax
from jax.experimental import pallas as pl
from jax.experimental.pallas import tpu as pltpu
from jax.experimental.pallas import tpu_sc as plsc

DIM = 64
NC, NS = 2, 16
NW = NC * NS


def _sc_gather(table, center_idx, context_idx):
    batch = center_idx.shape[0]
    bpw = batch // NW
    mesh = plsc.VectorSubcoreMesh(core_axis_name="c", subcore_axis_name="s")

    @functools.partial(
        pl.kernel,
        mesh=mesh,
        compiler_params=pltpu.CompilerParams(use_tc_tiling_on_sc=False),
        out_type=[jax.ShapeDtypeStruct((batch, DIM), jnp.float32),
                  jax.ShapeDtypeStruct((batch, DIM), jnp.float32)],
        scratch_types=[
            pltpu.VMEM((bpw,), jnp.int32),
            pltpu.VMEM((bpw, DIM), jnp.float32),
            pltpu.VMEM((bpw,), jnp.int32),
            pltpu.VMEM((bpw, DIM), jnp.float32),
            pltpu.SemaphoreType.DMA,
            pltpu.SemaphoreType.DMA,
        ],
    )
    def k(table_hbm, cen_hbm, ctx_hbm, out_cen_hbm, out_ctx_hbm,
          cen_idx_v, cen_rows_v, ctx_idx_v, ctx_rows_v, sem1, sem2):
        wid = lax.axis_index("s") * NC + lax.axis_index("c")
        base = wid * bpw
        pltpu.sync_copy(cen_hbm.at[pl.ds(base, bpw)], cen_idx_v)
        pltpu.sync_copy(ctx_hbm.at[pl.ds(base, bpw)], ctx_idx_v)
        c1 = pltpu.async_copy(table_hbm.at[cen_idx_v], cen_rows_v, sem1)
        c2 = pltpu.async_copy(table_hbm.at[ctx_idx_v], ctx_rows_v, sem2)
        c1.wait()
        c2.wait()
        pltpu.sync_copy(cen_rows_v, out_cen_hbm.at[pl.ds(base, bpw)])
        pltpu.sync_copy(ctx_rows_v, out_ctx_hbm.at[pl.ds(base, bpw)])

    return k(table, center_idx, context_idx)


def _tc_cosine_body(c_ref, x_ref, o_ref):
    c = c_ref[...]
    x = x_ref[...]
    dot = jnp.sum(c * x, axis=1)
    cn = jnp.sqrt(jnp.sum(c * c, axis=1))
    xn = jnp.sqrt(jnp.sum(x * x, axis=1))
    o_ref[...] = dot / (cn * xn + 1e-8)


def _tc_cosine(center_embed, context_embed):
    batch = center_embed.shape[0]
    return pl.pallas_call(
        _tc_cosine_body,
        out_shape=jax.ShapeDtypeStruct((batch,), jnp.float32),
    )(center_embed, context_embed)


@jax.jit
def kernel(center_idx, context_idx, table):
    ce, xe = _sc_gather(table,
                        center_idx.astype(jnp.int32),
                        context_idx.astype(jnp.int32))
    return _tc_cosine(ce, xe)
